# baseline (device time: 11000 ns/iter reference)
import jax
import jax.numpy as jnp
from jax import lax
from jax.experimental import pallas as pl
from jax.experimental.pallas import tpu as pltpu


def kernel(x, dy, gamma):
    del gamma
    m_per, d = x.shape

    def body(x_ref, dy_ref, out_ref, partial_ref, recv_ref, send_sem, recv_sem):
        my_x = lax.axis_index("x")
        my_y = lax.axis_index("y")
        my_z = lax.axis_index("z")
        partner = (my_x, my_y, 1 - my_z)

        barrier_sem = pltpu.get_barrier_semaphore()
        pl.semaphore_signal(
            barrier_sem, inc=1,
            device_id=partner, device_id_type=pl.DeviceIdType.MESH,
        )

        xv = x_ref[:, :]
        dyv = dy_ref[:, :]
        ones_col = jnp.ones((d, 1), jnp.float32)

        s = jnp.dot(xv, ones_col, preferred_element_type=jnp.float32)
        s2 = jnp.dot(xv * xv, ones_col, preferred_element_type=jnp.float32)
        mu = s * (1.0 / d)
        var = s2 * (1.0 / d) - mu * mu
        rstd = lax.rsqrt(var + 1e-5)

        dims = (((0,), (0,)), ((), ()))
        b = jnp.concatenate([rstd * mu, ones_col[:m_per]], axis=1)
        c = lax.dot_general(b, dyv, dims,
                            preferred_element_type=jnp.float32)
        g = lax.dot_general(rstd, dyv * xv, dims,
                            preferred_element_type=jnp.float32)
        dgamma = g - c[0:1, :]
        dbeta = c[1:2, :]
        partial_ref[:, :] = jnp.concatenate([dgamma, dbeta], axis=0)

        pl.semaphore_wait(barrier_sem, 1)

        rdma = pltpu.make_async_remote_copy(
            src_ref=partial_ref,
            dst_ref=recv_ref,
            send_sem=send_sem,
            recv_sem=recv_sem,
            device_id=partner,
            device_id_type=pl.DeviceIdType.MESH,
        )
        rdma.start()
        rdma.wait()

        out_ref[:, :] = partial_ref[:, :] + recv_ref[:, :]

    return pl.pallas_call(
        body,
        out_shape=jax.ShapeDtypeStruct((2, d), jnp.float32),
        in_specs=[
            pl.BlockSpec(memory_space=pltpu.VMEM),
            pl.BlockSpec(memory_space=pltpu.VMEM),
        ],
        out_specs=pl.BlockSpec(memory_space=pltpu.VMEM),
        scratch_shapes=[
            pltpu.VMEM((2, d), jnp.float32),
            pltpu.VMEM((2, d), jnp.float32),
            pltpu.SemaphoreType.DMA,
            pltpu.SemaphoreType.DMA,
        ],
        compiler_params=pltpu.CompilerParams(collective_id=0),
    )(x, dy)
